# baseline (speedup 1.0000x reference)
import functools

import jax
import jax.numpy as jnp
import numpy as np
from jax import lax
from jax.experimental import pallas as pl
from jax.experimental.pallas import tpu as pltpu
from jax.experimental.pallas import tpu_sc as plsc

_N = 10000
_E = 320000
_NC = 2
_NS = 16
_NW = _NC * _NS
_EPW = _E // _NW
_B = 40
_CH = _EPW // _B
_R = 6
_ROFF = 624
_RSZ = 640

def _bitrev3(h):
    return ((h & 1) << 2) | (h & 2) | ((h >> 2) & 1)


_PERM = np.zeros(128, np.int32)
for _j in range(8):
    for _m in range(8):
        for _r in range(2):
            _PERM[16 * _j + 2 * _m + _r] = _bitrev3(_m) * 16 + 2 * _j + _r
_RP1 = np.zeros((16, 128), np.float32)
for _c in range(128):
    _RP1[(_c % 16) & ~1, _c] = 1.0


def _make_edge_kernel(F, H):
    C = F // H
    F2 = F + 16
    KH = F // 16

    mesh = plsc.VectorSubcoreMesh(core_axis_name="c", subcore_axis_name="s")

    @functools.partial(
        pl.kernel,
        mesh=mesh,
        compiler_params=pltpu.CompilerParams(use_tc_tiling_on_sc=False),
        out_type=jax.ShapeDtypeStruct((_NC, _N, F2), jnp.float32),
        scratch_types=[
            pltpu.VMEM((_R, 2, _B), jnp.int32),
            pltpu.VMEM((_B, F), jnp.float32),
            pltpu.VMEM((_B, F), jnp.float32),
            pltpu.VMEM((_B, F), jnp.float32),
            pltpu.VMEM((_B, F), jnp.float32),
            pltpu.VMEM((_B, F2), jnp.float32),
            pltpu.VMEM((_B, F2), jnp.float32),
            pltpu.VMEM((H, C), jnp.float32),
            pltpu.VMEM_SHARED((_N, F2), jnp.float32),
            pltpu.SemaphoreType.DMA((_R,)),
            pltpu.SemaphoreType.DMA,
            pltpu.SemaphoreType.DMA,
            pltpu.SemaphoreType.DMA,
            pltpu.SemaphoreType.DMA,
            pltpu.SemaphoreType.DMA,
            pltpu.SemaphoreType.DMA,
        ],
    )
    def edge_kernel(xl_hbm, xr_hbm, idx_hbm, att_hbm, z_hbm, out_hbm,
                    ring, xlb0, xlb1, xrb0, xrb1, mb0, mb1, attv, acc,
                    isem, gx0, gx1, gr0, gr1, sc0, sc1):
        cid = lax.axis_index("c")
        sid = lax.axis_index("s")
        wid = sid * _NC + cid

        xlbs = (xlb0, xlb1)
        xrbs = (xrb0, xrb1)
        mbs = (mb0, mb1)
        gxs = (gx0, gx1)
        grs = (gr0, gr1)
        scs = (sc0, sc1)

        pltpu.sync_copy(att_hbm, attv)
        pltpu.sync_copy(z_hbm.at[pl.ds(sid * _ROFF, _RSZ), :],
                        acc.at[pl.ds(sid * _ROFF, _RSZ), :])
        plsc.subcore_barrier()

        lanes = lax.iota(jnp.int32, 16)
        att_vecs = [attv[(16 * k) // C, pl.ds((16 * k) % C, 16)]
                    for k in range(KH)]

        def issue_i(k, slot):
            pltpu.async_copy(idx_hbm.at[wid, k], ring.at[slot], isem.at[slot])

        def wait_i(k, slot):
            pltpu.make_async_copy(idx_hbm.at[wid, k], ring.at[slot],
                                  isem.at[slot]).wait()

        def issue_g(slot, buf):
            pltpu.async_copy(xl_hbm.at[ring.at[slot, 0]], xlbs[buf], gxs[buf])
            pltpu.async_copy(xr_hbm.at[ring.at[slot, 1]], xrbs[buf], grs[buf])

        def wait_g(slot, buf):
            pltpu.make_async_copy(xl_hbm.at[ring.at[slot, 0]], xlbs[buf],
                                  gxs[buf]).wait()
            pltpu.make_async_copy(xr_hbm.at[ring.at[slot, 1]], xrbs[buf],
                                  grs[buf]).wait()

        def issue_s(slot, buf):
            pltpu.async_copy(mbs[buf], acc.at[ring.at[slot, 1]], scs[buf],
                             add=True)

        def wait_s(slot, buf):
            pltpu.make_async_copy(mbs[buf], acc.at[ring.at[slot, 1]],
                                  scs[buf]).wait()

        def compute(buf):
            xlb = xlbs[buf]
            xrb = xrbs[buf]
            mb = mbs[buf]

            def edge_body(e, c2):
                xs = []
                ws = []
                for kk in range(KH):
                    xv = xlb[e, pl.ds(16 * kk, 16)]
                    rv = xrb[e, pl.ds(16 * kk, 16)]
                    xs.append(xv)
                    s = xv + rv
                    t = jnp.maximum(s, 0.2 * s)
                    ws.append(t * att_vecs[kk])
                tot = ws[0]
                for kk in range(1, KH):
                    tot = tot + ws[kk]
                if H == 1:
                    for m in (8, 4, 2):
                        tot = tot + jnp.take(tot, lanes ^ m)
                v = tot + jnp.take(tot, lanes ^ 1)
                pf = jnp.exp(v)
                mb[e, pl.ds(F, 16)] = pf
                for kk in range(KH):
                    mb[e, pl.ds(16 * kk, 16)] = xs[kk] * pf
                return c2

            lax.fori_loop(0, _B, edge_body, 0, unroll=4)

        def step(k, kdyn):
            slot = k % _R
            buf = k % 2
            wait_i(kdyn + 1, (k + 1) % _R)
            issue_g((k + 1) % _R, 1 - buf)
            wait_g(slot, buf)
            if k >= 2:
                wait_s((k - 2) % _R, buf)
            issue_i(kdyn + 4, (k + 4) % _R)
            compute(buf)
            issue_s(slot, buf)

        def tail_step(k):
            slot = k % _R
            buf = k % 2
            if k + 1 < _CH:
                wait_i(k + 1, (k + 1) % _R)
                issue_g((k + 1) % _R, 1 - buf)
            wait_g(slot, buf)
            wait_s((k - 2) % _R, buf)
            compute(buf)
            issue_s(slot, buf)

        for k in range(4):
            issue_i(k, k)
        wait_i(0, 0)
        issue_g(0, 0)
        for k in range(6):
            step(k, k)

        def hex_body(i, carry):
            k0 = 6 * i
            for o in range(6):
                step(6 + o, k0 + o)
            return carry

        lax.fori_loop(1, (_CH - 4) // 6, hex_body, 0)
        for k in range(_CH - 4, _CH):
            tail_step(k)
        wait_s((_CH - 2) % _R, 0)
        wait_s((_CH - 1) % _R, 1)

        plsc.subcore_barrier()
        pltpu.sync_copy(acc.at[pl.ds(sid * _ROFF, _RSZ), :],
                        out_hbm.at[cid, pl.ds(sid * _ROFF, _RSZ), :])

    return edge_kernel


_edge_l1 = _make_edge_kernel(128, 8)
_edge_l2 = _make_edge_kernel(64, 1)


_BLK = 2000
_GRID = _N // _BLK


def _proj_body(x_ref, wl_ref, bl_ref, wr_ref, br_ref, xl_ref, xr_ref):
    xb = x_ref[...]
    dn = (((1,), (1,)), ((), ()))
    xl_ref[...] = lax.dot_general(xb, wl_ref[...], dn,
                                  preferred_element_type=jnp.float32) + bl_ref[...]
    xr_ref[...] = lax.dot_general(xb, wr_ref[...], dn,
                                  preferred_element_type=jnp.float32) + br_ref[...]


_proj = pl.pallas_call(
    _proj_body,
    grid=(_GRID,),
    in_specs=[
        pl.BlockSpec((_BLK, 128), lambda i: (i, 0)),
        pl.BlockSpec((128, 128), lambda i: (0, 0)),
        pl.BlockSpec((1, 128), lambda i: (0, 0)),
        pl.BlockSpec((128, 128), lambda i: (0, 0)),
        pl.BlockSpec((1, 128), lambda i: (0, 0)),
    ],
    out_specs=[pl.BlockSpec((_BLK, 128), lambda i: (i, 0)),
               pl.BlockSpec((_BLK, 128), lambda i: (i, 0))],
    out_shape=[jax.ShapeDtypeStruct((_N, 128), jnp.float32)] * 2,
)


def _combine1_body(a0_ref, a1_ref, r1_ref, b1_ref, wl_ref, bl_ref,
                   wr_ref, br_ref, xl_ref, xr_ref):
    acc = a0_ref[...] + a1_ref[...]
    num = acc[:, :128]
    den16 = acc[:, 128:144]
    d = jnp.dot(den16, r1_ref[...], preferred_element_type=jnp.float32)
    avg = num / (d + 1e-30)
    y = avg + b1_ref[...]
    h = jnp.where(y > 0, y, jnp.exp(jnp.minimum(y, 0.0)) - 1.0)
    dn = (((1,), (1,)), ((), ()))
    xl_ref[...] = lax.dot_general(h, wl_ref[...], dn,
                                  preferred_element_type=jnp.float32) + bl_ref[...]
    xr_ref[...] = lax.dot_general(h, wr_ref[...], dn,
                                  preferred_element_type=jnp.float32) + br_ref[...]


_combine1 = pl.pallas_call(
    _combine1_body,
    grid=(_GRID,),
    in_specs=[
        pl.BlockSpec((_BLK, 144), lambda i: (i, 0)),
        pl.BlockSpec((_BLK, 144), lambda i: (i, 0)),
        pl.BlockSpec((16, 128), lambda i: (0, 0)),
        pl.BlockSpec((1, 128), lambda i: (0, 0)),
        pl.BlockSpec((64, 128), lambda i: (0, 0)),
        pl.BlockSpec((1, 64), lambda i: (0, 0)),
        pl.BlockSpec((64, 128), lambda i: (0, 0)),
        pl.BlockSpec((1, 64), lambda i: (0, 0)),
    ],
    out_specs=[pl.BlockSpec((_BLK, 64), lambda i: (i, 0)),
               pl.BlockSpec((_BLK, 64), lambda i: (i, 0))],
    out_shape=[jax.ShapeDtypeStruct((_N, 64), jnp.float32)] * 2,
)


def _final_body(a0_ref, a1_ref, r2_ref, b2_ref, out_ref):
    acc = a0_ref[...] + a1_ref[...]
    num = acc[:, :64]
    den8 = acc[:, 64:72]
    d = jnp.dot(den8, r2_ref[...], preferred_element_type=jnp.float32)
    out_ref[...] = num / (d + 1e-30) + b2_ref[...]


_final = pl.pallas_call(
    _final_body,
    grid=(_GRID,),
    in_specs=[
        pl.BlockSpec((_BLK, 80), lambda i: (i, 0)),
        pl.BlockSpec((_BLK, 80), lambda i: (i, 0)),
        pl.BlockSpec((8, 64), lambda i: (0, 0)),
        pl.BlockSpec((1, 64), lambda i: (0, 0)),
    ],
    out_specs=pl.BlockSpec((_BLK, 64), lambda i: (i, 0)),
    out_shape=jax.ShapeDtypeStruct((_N, 64), jnp.float32),
)


def kernel(args, x, edge_index, Wl1, bl1, Wr1, br1, att1, bias1,
           Wl2, bl2, Wr2, br2, att2, bias2):
    idx2 = edge_index.reshape(2, _NW, _CH, _B).transpose(1, 2, 0, 3)
    perm = jnp.asarray(_PERM)

    xl1, xr1 = _proj(x, Wl1[perm], bl1[perm].reshape(1, -1),
                     Wr1[perm], br1[perm].reshape(1, -1))

    z1 = jnp.zeros((_N, 144), jnp.float32)
    att1p = att1.reshape(-1)[perm].reshape(8, 16)
    acc1 = _edge_l1(xl1, xr1, idx2, att1p, z1)

    r1 = jnp.asarray(_RP1)
    xl2, xr2 = _combine1(acc1[0], acc1[1], r1, bias1[perm].reshape(1, -1),
                         Wl2[:, perm], bl2.reshape(1, -1),
                         Wr2[:, perm], br2.reshape(1, -1))

    z2 = jnp.zeros((_N, 80), jnp.float32)
    acc2 = _edge_l2(xl2, xr2, idx2, att2, z2)

    r2 = jnp.zeros((8, 64), jnp.float32).at[0].set(1.0)
    out = _final(acc2[0], acc2[1], r2, bias2.reshape(1, -1))
    return out


# speedup vs baseline: 1.0429x; 1.0429x over previous
import functools

import jax
import jax.numpy as jnp
import numpy as np
from jax import lax
from jax.experimental import pallas as pl
from jax.experimental.pallas import tpu as pltpu
from jax.experimental.pallas import tpu_sc as plsc

_N = 10000
_E = 320000
_NC = 2
_NS = 16
_NW = _NC * _NS
_EPW = _E // _NW
_B = 40
_CH = _EPW // _B
_R = 6
_ROFF = 624
_RSZ = 640

def _bitrev3(h):
    return ((h & 1) << 2) | (h & 2) | ((h >> 2) & 1)


_PERM = np.zeros(128, np.int32)
for _j in range(8):
    for _m in range(8):
        for _r in range(2):
            _PERM[16 * _j + 2 * _m + _r] = _bitrev3(_m) * 16 + 2 * _j + _r
_RP1 = np.zeros((16, 128), np.float32)
for _c in range(128):
    _RP1[(_c % 16) & ~1, _c] = 1.0


def _make_edge_kernel(F, H):
    C = F // H
    F2 = F + 16
    KH = F // 16

    mesh = plsc.VectorSubcoreMesh(core_axis_name="c", subcore_axis_name="s")

    @functools.partial(
        pl.kernel,
        mesh=mesh,
        compiler_params=pltpu.CompilerParams(use_tc_tiling_on_sc=False),
        out_type=jax.ShapeDtypeStruct((_NC, _N, F2), jnp.float32),
        scratch_types=[
            pltpu.VMEM((_R, _B), jnp.int32),
            pltpu.VMEM((_R, _B), jnp.int32),
            pltpu.VMEM((_B, F), jnp.float32),
            pltpu.VMEM((_B, F), jnp.float32),
            pltpu.VMEM((_B, F), jnp.float32),
            pltpu.VMEM((_B, F), jnp.float32),
            pltpu.VMEM((_B, F2), jnp.float32),
            pltpu.VMEM((_B, F2), jnp.float32),
            pltpu.VMEM((H, C), jnp.float32),
            pltpu.VMEM_SHARED((_N, F2), jnp.float32),
            pltpu.SemaphoreType.DMA((_R,)),
            pltpu.SemaphoreType.DMA,
            pltpu.SemaphoreType.DMA,
            pltpu.SemaphoreType.DMA,
            pltpu.SemaphoreType.DMA,
            pltpu.SemaphoreType.DMA,
            pltpu.SemaphoreType.DMA,
        ],
    )
    def edge_kernel(xl_hbm, xr_hbm, src_hbm, dst_hbm, att_hbm, z_hbm, out_hbm,
                    sring, dring, xlb0, xlb1, xrb0, xrb1, mb0, mb1, attv, acc,
                    isem, gx0, gx1, gr0, gr1, sc0, sc1):
        cid = lax.axis_index("c")
        sid = lax.axis_index("s")
        wid = sid * _NC + cid

        xlbs = (xlb0, xlb1)
        xrbs = (xrb0, xrb1)
        mbs = (mb0, mb1)
        gxs = (gx0, gx1)
        grs = (gr0, gr1)
        scs = (sc0, sc1)

        pltpu.sync_copy(att_hbm, attv)
        pltpu.sync_copy(z_hbm, acc.at[pl.ds(sid * _ROFF, _RSZ), :])
        plsc.subcore_barrier()

        lanes = lax.iota(jnp.int32, 16)
        att_vecs = [attv[(16 * k) // C, pl.ds((16 * k) % C, 16)]
                    for k in range(KH)]

        def issue_i(k, slot):
            pltpu.async_copy(src_hbm.at[wid, k], sring.at[slot], isem.at[slot])
            pltpu.async_copy(dst_hbm.at[wid, k], dring.at[slot], isem.at[slot])

        def wait_i(k, slot):
            pltpu.make_async_copy(src_hbm.at[wid, k], sring.at[slot],
                                  isem.at[slot]).wait()
            pltpu.make_async_copy(dst_hbm.at[wid, k], dring.at[slot],
                                  isem.at[slot]).wait()

        def issue_g(slot, buf):
            pltpu.async_copy(xl_hbm.at[sring.at[slot]], xlbs[buf], gxs[buf])
            pltpu.async_copy(xr_hbm.at[dring.at[slot]], xrbs[buf], grs[buf])

        def wait_g(slot, buf):
            pltpu.make_async_copy(xl_hbm.at[sring.at[slot]], xlbs[buf],
                                  gxs[buf]).wait()
            pltpu.make_async_copy(xr_hbm.at[dring.at[slot]], xrbs[buf],
                                  grs[buf]).wait()

        def issue_s(slot, buf):
            pltpu.async_copy(mbs[buf], acc.at[dring.at[slot]], scs[buf],
                             add=True)

        def wait_s(slot, buf):
            pltpu.make_async_copy(mbs[buf], acc.at[dring.at[slot]],
                                  scs[buf]).wait()

        def compute(buf):
            xlb = xlbs[buf]
            xrb = xrbs[buf]
            mb = mbs[buf]

            def edge_body(e, c2):
                xs = []
                ws = []
                for kk in range(KH):
                    xv = xlb[e, pl.ds(16 * kk, 16)]
                    rv = xrb[e, pl.ds(16 * kk, 16)]
                    xs.append(xv)
                    s = xv + rv
                    t = jnp.maximum(s, 0.2 * s)
                    ws.append(t * att_vecs[kk])
                tot = ws[0]
                for kk in range(1, KH):
                    tot = tot + ws[kk]
                if H == 1:
                    for m in (8, 4, 2):
                        tot = tot + jnp.take(tot, lanes ^ m)
                v = tot + jnp.take(tot, lanes ^ 1)
                pf = jnp.exp(v)
                mb[e, pl.ds(F, 16)] = pf
                for kk in range(KH):
                    mb[e, pl.ds(16 * kk, 16)] = xs[kk] * pf
                return c2

            lax.fori_loop(0, _B, edge_body, 0, unroll=8)

        def step(k, kdyn):
            slot = k % _R
            buf = k % 2
            wait_i(kdyn + 1, (k + 1) % _R)
            issue_g((k + 1) % _R, 1 - buf)
            wait_g(slot, buf)
            if k >= 2:
                wait_s((k - 2) % _R, buf)
            issue_i(kdyn + 4, (k + 4) % _R)
            compute(buf)
            issue_s(slot, buf)

        def tail_step(k):
            slot = k % _R
            buf = k % 2
            if k + 1 < _CH:
                wait_i(k + 1, (k + 1) % _R)
                issue_g((k + 1) % _R, 1 - buf)
            wait_g(slot, buf)
            wait_s((k - 2) % _R, buf)
            compute(buf)
            issue_s(slot, buf)

        for k in range(4):
            issue_i(k, k)
        wait_i(0, 0)
        issue_g(0, 0)
        for k in range(6):
            step(k, k)

        def hex_body(i, carry):
            k0 = 6 * i
            for o in range(6):
                step(6 + o, k0 + o)
            return carry

        lax.fori_loop(1, (_CH - 4) // 6, hex_body, 0)
        for k in range(_CH - 4, _CH):
            tail_step(k)
        wait_s((_CH - 2) % _R, 0)
        wait_s((_CH - 1) % _R, 1)

        plsc.subcore_barrier()
        pltpu.sync_copy(acc.at[pl.ds(sid * _ROFF, _RSZ), :],
                        out_hbm.at[cid, pl.ds(sid * _ROFF, _RSZ), :])

    return edge_kernel


_edge_l1 = _make_edge_kernel(128, 8)
_edge_l2 = _make_edge_kernel(64, 1)


_BLK = 2000
_GRID = _N // _BLK


def _proj_body(x_ref, wl_ref, bl_ref, wr_ref, br_ref, xl_ref, xr_ref):
    xb = x_ref[...]
    dn = (((1,), (1,)), ((), ()))
    xl_ref[...] = lax.dot_general(xb, wl_ref[...], dn,
                                  preferred_element_type=jnp.float32) + bl_ref[...]
    xr_ref[...] = lax.dot_general(xb, wr_ref[...], dn,
                                  preferred_element_type=jnp.float32) + br_ref[...]


_proj = pl.pallas_call(
    _proj_body,
    grid=(_GRID,),
    in_specs=[
        pl.BlockSpec((_BLK, 128), lambda i: (i, 0)),
        pl.BlockSpec((128, 128), lambda i: (0, 0)),
        pl.BlockSpec((1, 128), lambda i: (0, 0)),
        pl.BlockSpec((128, 128), lambda i: (0, 0)),
        pl.BlockSpec((1, 128), lambda i: (0, 0)),
    ],
    out_specs=[pl.BlockSpec((_BLK, 128), lambda i: (i, 0)),
               pl.BlockSpec((_BLK, 128), lambda i: (i, 0))],
    out_shape=[jax.ShapeDtypeStruct((_N, 128), jnp.float32)] * 2,
)


def _combine1_body(a_ref, r1_ref, b1_ref, wl_ref, bl_ref,
                   wr_ref, br_ref, xl_ref, xr_ref):
    acc = a_ref[0, :, :] + a_ref[1, :, :]
    num = acc[:, :128]
    den16 = acc[:, 128:144]
    d = jnp.dot(den16, r1_ref[...], preferred_element_type=jnp.float32)
    avg = num / (d + 1e-30)
    y = avg + b1_ref[...]
    h = jnp.where(y > 0, y, jnp.exp(jnp.minimum(y, 0.0)) - 1.0)
    dn = (((1,), (1,)), ((), ()))
    xl_ref[...] = lax.dot_general(h, wl_ref[...], dn,
                                  preferred_element_type=jnp.float32) + bl_ref[...]
    xr_ref[...] = lax.dot_general(h, wr_ref[...], dn,
                                  preferred_element_type=jnp.float32) + br_ref[...]


_combine1 = pl.pallas_call(
    _combine1_body,
    grid=(_GRID,),
    in_specs=[
        pl.BlockSpec((2, _BLK, 144), lambda i: (0, i, 0)),
        pl.BlockSpec((16, 128), lambda i: (0, 0)),
        pl.BlockSpec((1, 128), lambda i: (0, 0)),
        pl.BlockSpec((64, 128), lambda i: (0, 0)),
        pl.BlockSpec((1, 64), lambda i: (0, 0)),
        pl.BlockSpec((64, 128), lambda i: (0, 0)),
        pl.BlockSpec((1, 64), lambda i: (0, 0)),
    ],
    out_specs=[pl.BlockSpec((_BLK, 64), lambda i: (i, 0)),
               pl.BlockSpec((_BLK, 64), lambda i: (i, 0))],
    out_shape=[jax.ShapeDtypeStruct((_N, 64), jnp.float32)] * 2,
)


def _final_body(a_ref, r2_ref, b2_ref, out_ref):
    acc = a_ref[0, :, :] + a_ref[1, :, :]
    num = acc[:, :64]
    den8 = acc[:, 64:72]
    d = jnp.dot(den8, r2_ref[...], preferred_element_type=jnp.float32)
    out_ref[...] = num / (d + 1e-30) + b2_ref[...]


_final = pl.pallas_call(
    _final_body,
    grid=(_GRID,),
    in_specs=[
        pl.BlockSpec((2, _BLK, 80), lambda i: (0, i, 0)),
        pl.BlockSpec((8, 64), lambda i: (0, 0)),
        pl.BlockSpec((1, 64), lambda i: (0, 0)),
    ],
    out_specs=pl.BlockSpec((_BLK, 64), lambda i: (i, 0)),
    out_shape=jax.ShapeDtypeStruct((_N, 64), jnp.float32),
)


def kernel(args, x, edge_index, Wl1, bl1, Wr1, br1, att1, bias1,
           Wl2, bl2, Wr2, br2, att2, bias2):
    src = edge_index[0].reshape(_NW, _CH, _B)
    dst = edge_index[1].reshape(_NW, _CH, _B)
    perm = jnp.asarray(_PERM)

    xl1, xr1 = _proj(x, Wl1[perm], bl1[perm].reshape(1, -1),
                     Wr1[perm], br1[perm].reshape(1, -1))

    z1 = jnp.zeros((_RSZ, 144), jnp.float32)
    att1p = att1.reshape(-1)[perm].reshape(8, 16)
    acc1 = _edge_l1(xl1, xr1, src, dst, att1p, z1)

    r1 = jnp.asarray(_RP1)
    xl2, xr2 = _combine1(acc1, r1, bias1[perm].reshape(1, -1),
                         Wl2[:, perm], bl2.reshape(1, -1),
                         Wr2[:, perm], br2.reshape(1, -1))

    z2 = jnp.zeros((_RSZ, 80), jnp.float32)
    acc2 = _edge_l2(xl2, xr2, src, dst, att2, z2)

    r2 = jnp.zeros((8, 64), jnp.float32).at[0].set(1.0)
    out = _final(acc2, r2, bias2.reshape(1, -1))
    return out
